# baseline (speedup 1.0000x reference)
import functools

import jax
import jax.numpy as jnp
from jax.experimental import pallas as pl
from jax.experimental.pallas import tpu as pltpu

_BB = 8
_G = 128
_WSUB = 16


def _diff_conv_body(n, din, dout, k_hops, bb, g_unroll, wsub,
                    a_ref, x_ref, ones_ref, p_ref, o_ref, xcat_ref):
    bbn = bb * n
    wn = wsub * n
    f32 = jnp.float32
    bf16 = jnp.bfloat16
    ones_bd = ones_ref[...]
    wbd = p_ref[0:wn, 0:wn]
    lw = p_ref[wn:wn + k_hops * din, 0:dout]
    lb = p_ref[wn + k_hops * din:wn + k_hops * din + 1, 0:dout]

    a = a_ref[...]
    rs = jnp.dot(a, ones_bd, preferred_element_type=f32)
    trans = (a.astype(f32) * pl.reciprocal(rs + 1e-12, approx=False)
             ).astype(bf16)
    e = jnp.concatenate(
        [jnp.exp(jnp.dot(wbd, trans[h * wn:(h + 1) * wn, :],
                         preferred_element_type=f32))
         for h in range(g_unroll // wsub)], axis=0)
    den = jnp.dot(e.astype(bf16), ones_bd, preferred_element_type=f32)
    t = (e * pl.reciprocal(den, approx=False)).astype(bf16)

    for q in range(g_unroll):
        tq = t[q * n:(q + 1) * n, :]
        t_bd = jnp.concatenate([tq] * bb, axis=0) * ones_bd
        xq = x_ref[q * bbn:(q + 1) * bbn, :].astype(bf16)
        xcat_ref[q * bbn:(q + 1) * bbn, 0:din] = xq
        x1b = jnp.dot(t_bd, xq, preferred_element_type=f32).astype(bf16)
        x2b = jnp.dot(t_bd, x1b, preferred_element_type=f32).astype(bf16)
        xcat_ref[q * bbn:(q + 1) * bbn, din:2 * din] = x1b
        xcat_ref[q * bbn:(q + 1) * bbn, 2 * din:3 * din] = x2b

    fsub = min(1024, g_unroll * bbn)
    for h in range((g_unroll * bbn) // fsub):
        acc = jnp.dot(xcat_ref[h * fsub:(h + 1) * fsub, :], lw,
                      preferred_element_type=f32)
        acc = (acc + lb) * (1.0 / k_hops)
        o_ref[h * fsub:(h + 1) * fsub, :] = jnp.maximum(acc, 0.0)


def kernel(X, A, W, lin_w, lin_b):
    f32 = jnp.float32
    bf16 = jnp.bfloat16
    b, n, din = X.shape
    k_hops, _, dout = lin_w.shape
    bb = _BB
    bbn = bb * n
    c = b // bb
    g_unroll = min(_G, c)
    s = c // g_unroll

    X2 = X.reshape(b * n, din)

    A_cmp = (A.astype(bf16)
             .reshape(c, bb, n, n)
             .transpose(0, 2, 1, 3)
             .reshape(c * n, bbn))

    ones_bd = jnp.kron(jnp.eye(bb, dtype=bf16), jnp.ones((n, n), bf16))

    wsub = min(_WSUB, g_unroll)
    gn = wsub * n
    pl_w = max(gn, dout)
    w_bd = jnp.kron(jnp.eye(wsub, dtype=f32), W.reshape(n, n))
    lw = lin_w.reshape(k_hops * din, dout)
    lb = jnp.sum(lin_b, axis=0, keepdims=True)

    def pad_cols(m):
        return jnp.pad(m, ((0, 0), (0, pl_w - m.shape[1])))
    p_rows = gn + k_hops * din + 8
    params = jnp.concatenate(
        [pad_cols(w_bd), pad_cols(lw), pad_cols(lb),
         jnp.zeros((7, pl_w), f32)], axis=0).astype(bf16)

    body = functools.partial(_diff_conv_body, n, din, dout, k_hops, bb,
                             g_unroll, wsub)
    out2 = pl.pallas_call(
        body,
        out_shape=jax.ShapeDtypeStruct((b * n, dout), f32),
        grid=(s,),
        in_specs=[
            pl.BlockSpec((g_unroll * n, bbn), lambda i: (i, 0)),
            pl.BlockSpec((g_unroll * bbn, din), lambda i: (i, 0)),
            pl.BlockSpec((bbn, bbn), lambda i: (0, 0)),
            pl.BlockSpec((p_rows, pl_w), lambda i: (0, 0)),
        ],
        out_specs=pl.BlockSpec((g_unroll * bbn, dout), lambda i: (i, 0)),
        scratch_shapes=[
            pltpu.VMEM((g_unroll * bbn, k_hops * din), bf16)],
        compiler_params=pltpu.CompilerParams(
            dimension_semantics=("parallel",)),
    )(A_cmp, X2, ones_bd, params)
    return out2.reshape(b, n, dout)


# speedup vs baseline: 1.0259x; 1.0259x over previous
import functools

import jax
import jax.numpy as jnp
from jax.experimental import pallas as pl
from jax.experimental.pallas import tpu as pltpu

_BB = 8
_G = 32
_WSUB = 16


def _diff_conv_body(n, din, dout, k_hops, bb, g_unroll, wsub,
                    a_ref, x_ref, ones_ref, p_ref, o_ref, xcat_ref):
    bbn = bb * n
    wn = wsub * n
    f32 = jnp.float32
    bf16 = jnp.bfloat16
    ones_bd = ones_ref[...]
    wbd = p_ref[0:wn, 0:wn]
    lw = p_ref[wn:wn + k_hops * din, 0:dout]
    lb = p_ref[wn + k_hops * din:wn + k_hops * din + 1, 0:dout]

    a = a_ref[...]
    rs = jnp.dot(a, ones_bd, preferred_element_type=f32)
    trans = (a.astype(f32) * pl.reciprocal(rs + 1e-12, approx=False)
             ).astype(bf16)
    e = jnp.concatenate(
        [jnp.exp(jnp.dot(wbd, trans[h * wn:(h + 1) * wn, :],
                         preferred_element_type=f32))
         for h in range(g_unroll // wsub)], axis=0)
    den = jnp.dot(e.astype(bf16), ones_bd, preferred_element_type=f32)
    t = (e * pl.reciprocal(den, approx=False)).astype(bf16)

    for q in range(g_unroll):
        tq = t[q * n:(q + 1) * n, :]
        t_bd = jnp.concatenate([tq] * bb, axis=0) * ones_bd
        xq = x_ref[q * bbn:(q + 1) * bbn, :].astype(bf16)
        xcat_ref[q * bbn:(q + 1) * bbn, 0:din] = xq
        x1b = jnp.dot(t_bd, xq, preferred_element_type=f32).astype(bf16)
        x2b = jnp.dot(t_bd, x1b, preferred_element_type=f32).astype(bf16)
        xcat_ref[q * bbn:(q + 1) * bbn, din:2 * din] = x1b
        xcat_ref[q * bbn:(q + 1) * bbn, 2 * din:3 * din] = x2b

    fsub = min(1024, g_unroll * bbn)
    for h in range((g_unroll * bbn) // fsub):
        acc = jnp.dot(xcat_ref[h * fsub:(h + 1) * fsub, :], lw,
                      preferred_element_type=f32)
        acc = (acc + lb) * (1.0 / k_hops)
        o_ref[h * fsub:(h + 1) * fsub, :] = jnp.maximum(acc, 0.0)


def kernel(X, A, W, lin_w, lin_b):
    f32 = jnp.float32
    bf16 = jnp.bfloat16
    b, n, din = X.shape
    k_hops, _, dout = lin_w.shape
    bb = _BB
    bbn = bb * n
    c = b // bb
    g_unroll = min(_G, c)
    s = c // g_unroll

    X2 = X.reshape(b * n, din)

    A_cmp = (A.astype(bf16)
             .reshape(c, bb, n, n)
             .transpose(0, 2, 1, 3)
             .reshape(c * n, bbn))

    ones_bd = jnp.kron(jnp.eye(bb, dtype=bf16), jnp.ones((n, n), bf16))

    wsub = min(_WSUB, g_unroll)
    gn = wsub * n
    pl_w = max(gn, dout)
    w_bd = jnp.kron(jnp.eye(wsub, dtype=f32), W.reshape(n, n))
    lw = lin_w.reshape(k_hops * din, dout)
    lb = jnp.sum(lin_b, axis=0, keepdims=True)

    def pad_cols(m):
        return jnp.pad(m, ((0, 0), (0, pl_w - m.shape[1])))
    p_rows = gn + k_hops * din + 8
    params = jnp.concatenate(
        [pad_cols(w_bd), pad_cols(lw), pad_cols(lb),
         jnp.zeros((7, pl_w), f32)], axis=0).astype(bf16)

    body = functools.partial(_diff_conv_body, n, din, dout, k_hops, bb,
                             g_unroll, wsub)
    out2 = pl.pallas_call(
        body,
        out_shape=jax.ShapeDtypeStruct((b * n, dout), f32),
        grid=(s,),
        in_specs=[
            pl.BlockSpec((g_unroll * n, bbn), lambda i: (i, 0)),
            pl.BlockSpec((g_unroll * bbn, din), lambda i: (i, 0)),
            pl.BlockSpec((bbn, bbn), lambda i: (0, 0)),
            pl.BlockSpec((p_rows, pl_w), lambda i: (0, 0)),
        ],
        out_specs=pl.BlockSpec((g_unroll * bbn, dout), lambda i: (i, 0)),
        scratch_shapes=[
            pltpu.VMEM((g_unroll * bbn, k_hops * din), bf16)],
        compiler_params=pltpu.CompilerParams(
            dimension_semantics=("parallel",)),
    )(A_cmp, X2, ones_bd, params)
    return out2.reshape(b, n, dout)


# speedup vs baseline: 1.0413x; 1.0150x over previous
import functools

import jax
import jax.numpy as jnp
from jax.experimental import pallas as pl
from jax.experimental.pallas import tpu as pltpu

_BB = 8
_G = 64
_WSUB = 16


def _diff_conv_body(n, din, dout, k_hops, bb, g_unroll, wsub,
                    a_ref, x_ref, ones_ref, p_ref, o_ref, xcat_ref):
    bbn = bb * n
    wn = wsub * n
    f32 = jnp.float32
    bf16 = jnp.bfloat16
    ones_bd = ones_ref[...]
    wbd = p_ref[0:wn, 0:wn]
    lw = p_ref[wn:wn + k_hops * din, 0:dout]
    lb = p_ref[wn + k_hops * din:wn + k_hops * din + 1, 0:dout]

    a = a_ref[...]
    rs = jnp.dot(a, ones_bd, preferred_element_type=f32)
    trans = (a.astype(f32) * pl.reciprocal(rs + 1e-12, approx=False)
             ).astype(bf16)
    e = jnp.concatenate(
        [jnp.exp(jnp.dot(wbd, trans[h * wn:(h + 1) * wn, :],
                         preferred_element_type=f32))
         for h in range(g_unroll // wsub)], axis=0)
    den = jnp.dot(e.astype(bf16), ones_bd, preferred_element_type=f32)
    t = (e * pl.reciprocal(den, approx=False)).astype(bf16)

    for q in range(g_unroll):
        tq = t[q * n:(q + 1) * n, :]
        t_bd = jnp.concatenate([tq] * bb, axis=0) * ones_bd
        xq = x_ref[q * bbn:(q + 1) * bbn, :].astype(bf16)
        xcat_ref[q * bbn:(q + 1) * bbn, 0:din] = xq
        x1b = jnp.dot(t_bd, xq, preferred_element_type=f32).astype(bf16)
        x2b = jnp.dot(t_bd, x1b, preferred_element_type=f32).astype(bf16)
        xcat_ref[q * bbn:(q + 1) * bbn, din:2 * din] = x1b
        xcat_ref[q * bbn:(q + 1) * bbn, 2 * din:3 * din] = x2b

    fsub = min(1024, g_unroll * bbn)
    for h in range((g_unroll * bbn) // fsub):
        acc = jnp.dot(xcat_ref[h * fsub:(h + 1) * fsub, :], lw,
                      preferred_element_type=f32)
        acc = (acc + lb) * (1.0 / k_hops)
        o_ref[h * fsub:(h + 1) * fsub, :] = jnp.maximum(acc, 0.0)


def kernel(X, A, W, lin_w, lin_b):
    f32 = jnp.float32
    bf16 = jnp.bfloat16
    b, n, din = X.shape
    k_hops, _, dout = lin_w.shape
    bb = _BB
    bbn = bb * n
    c = b // bb
    g_unroll = min(_G, c)
    s = c // g_unroll

    X2 = X.reshape(b * n, din)

    A_cmp = (A.astype(bf16)
             .reshape(c, bb, n, n)
             .transpose(0, 2, 1, 3)
             .reshape(c * n, bbn))

    ones_bd = jnp.kron(jnp.eye(bb, dtype=bf16), jnp.ones((n, n), bf16))

    wsub = min(_WSUB, g_unroll)
    gn = wsub * n
    pl_w = max(gn, dout)
    w_bd = jnp.kron(jnp.eye(wsub, dtype=f32), W.reshape(n, n))
    lw = lin_w.reshape(k_hops * din, dout)
    lb = jnp.sum(lin_b, axis=0, keepdims=True)

    def pad_cols(m):
        return jnp.pad(m, ((0, 0), (0, pl_w - m.shape[1])))
    p_rows = gn + k_hops * din + 8
    params = jnp.concatenate(
        [pad_cols(w_bd), pad_cols(lw), pad_cols(lb),
         jnp.zeros((7, pl_w), f32)], axis=0).astype(bf16)

    body = functools.partial(_diff_conv_body, n, din, dout, k_hops, bb,
                             g_unroll, wsub)
    out2 = pl.pallas_call(
        body,
        out_shape=jax.ShapeDtypeStruct((b * n, dout), f32),
        grid=(s,),
        in_specs=[
            pl.BlockSpec((g_unroll * n, bbn), lambda i: (i, 0)),
            pl.BlockSpec((g_unroll * bbn, din), lambda i: (i, 0)),
            pl.BlockSpec((bbn, bbn), lambda i: (0, 0)),
            pl.BlockSpec((p_rows, pl_w), lambda i: (0, 0)),
        ],
        out_specs=pl.BlockSpec((g_unroll * bbn, dout), lambda i: (i, 0)),
        scratch_shapes=[
            pltpu.VMEM((g_unroll * bbn, k_hops * din), bf16)],
        compiler_params=pltpu.CompilerParams(
            dimension_semantics=("parallel",)),
    )(A_cmp, X2, ones_bd, params)
    return out2.reshape(b, n, dout)


# speedup vs baseline: 1.0443x; 1.0029x over previous
import functools

import jax
import jax.numpy as jnp
from jax.experimental import pallas as pl
from jax.experimental.pallas import tpu as pltpu

_BB = 8
_G = 64
_WSUB = 16


def _diff_conv_body(n, din, dout, k_hops, bb, g_unroll, wsub,
                    a_ref, x_ref, ones_ref, p_ref, o_ref, xcat_ref):
    bbn = bb * n
    wn = wsub * n
    f32 = jnp.float32
    bf16 = jnp.bfloat16
    ones_bd = ones_ref[...]
    wbd = p_ref[0:wn, 0:wn]
    lw = p_ref[wn:wn + k_hops * din, 0:dout]
    lb = p_ref[wn + k_hops * din:wn + k_hops * din + 1, 0:dout]

    a = a_ref[...]
    rs = jnp.dot(a, ones_bd, preferred_element_type=f32)
    trans = (a.astype(f32) * pl.reciprocal(rs + 1e-12, approx=False)
             ).astype(bf16)
    e = jnp.concatenate(
        [jnp.exp(jnp.dot(wbd, trans[h * wn:(h + 1) * wn, :],
                         preferred_element_type=f32))
         for h in range(g_unroll // wsub)], axis=0)
    den = jnp.dot(e.astype(bf16), ones_bd, preferred_element_type=f32)
    t = (e * pl.reciprocal(den, approx=False)).astype(bf16)

    zb = jnp.zeros((bbn, din), bf16)
    for p in range(g_unroll // 2):
        q0 = 2 * p
        q1 = q0 + 1
        t0 = t[q0 * n:(q0 + 1) * n, :]
        t1 = t[q1 * n:(q1 + 1) * n, :]
        tpair = jnp.concatenate(
            [jnp.concatenate([t0] * bb, axis=0) * ones_bd,
             jnp.concatenate([t1] * bb, axis=0) * ones_bd], axis=1)
        x0 = x_ref[q0 * bbn:(q0 + 1) * bbn, :].astype(bf16)
        x1 = x_ref[q1 * bbn:(q1 + 1) * bbn, :].astype(bf16)
        xcat_ref[q0 * bbn:(q0 + 1) * bbn, 0:din] = x0
        xcat_ref[q1 * bbn:(q1 + 1) * bbn, 0:din] = x1
        xd = jnp.concatenate(
            [jnp.concatenate([x0, zb], axis=1),
             jnp.concatenate([zb, x1], axis=1)], axis=0)
        h1 = jnp.dot(tpair, xd, preferred_element_type=f32).astype(bf16)
        xcat_ref[q0 * bbn:(q0 + 1) * bbn, din:2 * din] = h1[:, 0:din]
        xcat_ref[q1 * bbn:(q1 + 1) * bbn, din:2 * din] = h1[:, din:2 * din]
        xd2 = jnp.concatenate(
            [jnp.concatenate([h1[:, 0:din], zb], axis=1),
             jnp.concatenate([zb, h1[:, din:2 * din]], axis=1)], axis=0)
        h2 = jnp.dot(tpair, xd2, preferred_element_type=f32).astype(bf16)
        xcat_ref[q0 * bbn:(q0 + 1) * bbn, 2 * din:3 * din] = h2[:, 0:din]
        xcat_ref[q1 * bbn:(q1 + 1) * bbn, 2 * din:3 * din] = h2[:, din:2 * din]

    fsub = min(1024, g_unroll * bbn)
    for h in range((g_unroll * bbn) // fsub):
        acc = jnp.dot(xcat_ref[h * fsub:(h + 1) * fsub, :], lw,
                      preferred_element_type=f32)
        acc = (acc + lb) * (1.0 / k_hops)
        o_ref[h * fsub:(h + 1) * fsub, :] = jnp.maximum(acc, 0.0)


def kernel(X, A, W, lin_w, lin_b):
    f32 = jnp.float32
    bf16 = jnp.bfloat16
    b, n, din = X.shape
    k_hops, _, dout = lin_w.shape
    bb = _BB
    bbn = bb * n
    c = b // bb
    g_unroll = min(_G, c)
    s = c // g_unroll

    X2 = X.reshape(b * n, din)

    A_cmp = (A.astype(bf16)
             .reshape(c, bb, n, n)
             .transpose(0, 2, 1, 3)
             .reshape(c * n, bbn))

    ones_bd = jnp.kron(jnp.eye(bb, dtype=bf16), jnp.ones((n, n), bf16))

    wsub = min(_WSUB, g_unroll)
    gn = wsub * n
    pl_w = max(gn, dout)
    w_bd = jnp.kron(jnp.eye(wsub, dtype=f32), W.reshape(n, n))
    lw = lin_w.reshape(k_hops * din, dout)
    lb = jnp.sum(lin_b, axis=0, keepdims=True)

    def pad_cols(m):
        return jnp.pad(m, ((0, 0), (0, pl_w - m.shape[1])))
    p_rows = gn + k_hops * din + 8
    params = jnp.concatenate(
        [pad_cols(w_bd), pad_cols(lw), pad_cols(lb),
         jnp.zeros((7, pl_w), f32)], axis=0).astype(bf16)

    body = functools.partial(_diff_conv_body, n, din, dout, k_hops, bb,
                             g_unroll, wsub)
    out2 = pl.pallas_call(
        body,
        out_shape=jax.ShapeDtypeStruct((b * n, dout), f32),
        grid=(s,),
        in_specs=[
            pl.BlockSpec((g_unroll * n, bbn), lambda i: (i, 0)),
            pl.BlockSpec((g_unroll * bbn, din), lambda i: (i, 0)),
            pl.BlockSpec((bbn, bbn), lambda i: (0, 0)),
            pl.BlockSpec((p_rows, pl_w), lambda i: (0, 0)),
        ],
        out_specs=pl.BlockSpec((g_unroll * bbn, dout), lambda i: (i, 0)),
        scratch_shapes=[
            pltpu.VMEM((g_unroll * bbn, k_hops * din), bf16)],
        compiler_params=pltpu.CompilerParams(
            dimension_semantics=("parallel",)),
    )(A_cmp, X2, ones_bd, params)
    return out2.reshape(b, n, dout)
